# baseline (speedup 1.0000x reference)
import jax
import jax.numpy as jnp
from jax import lax
from jax.experimental import pallas as pl
from jax.experimental.pallas import tpu as pltpu

_B, _H, _W = 4, 224, 224
_NEG = -1e30
_BIGI = 1 << 30


def _shift(a, s, axis, fill):
    b, h, w = a.shape
    if axis == 1:
        pad_shape = (b, abs(s), w)
    else:
        pad_shape = (b, h, abs(s))
    pad = jnp.full(pad_shape, fill, a.dtype)
    if axis == 1:
        if s > 0:
            return jnp.concatenate([a[:, s:, :], pad], axis=1)
        return jnp.concatenate([pad, a[:, :s, :]], axis=1)
    else:
        if s > 0:
            return jnp.concatenate([a[:, :, s:], pad], axis=2)
        return jnp.concatenate([pad, a[:, :, :s]], axis=2)


def _nms_kernel(x_ref, out_ref, v_ref, keep_ref):
    x = x_ref[...]

    xr = jnp.concatenate([x[:, :, 1:], x[:, :, -1:]], axis=2)
    xl = jnp.concatenate([x[:, :, :1], x[:, :, :-1]], axis=2)
    xd = jnp.concatenate([x[:, 1:, :], x[:, -1:, :]], axis=1)
    xu = jnp.concatenate([x[:, :1, :], x[:, :-1, :]], axis=1)
    rg_x = xr > x
    ll_x = x <= xl
    rg_y = xd > x
    ll_y = x <= xu
    neg = x <= 0
    valleys = rg_x & ll_x & rg_y & ll_y & neg
    peaks = (~rg_x) & (~ll_x) & (~rg_y) & (~ll_y) & (~neg)
    ext = peaks | valleys

    v_ref[...] = jnp.where(ext, jnp.abs(x), _NEG)
    keep_ref[...] = jnp.zeros_like(x)

    ri = lax.broadcasted_iota(jnp.int32, (_B, _H, _W), 1)
    ci = lax.broadcasted_iota(jnp.int32, (_B, _H, _W), 2)
    idx0 = ri * _W + ci

    def run_round():
        v = v_ref[...]
        mv, mi = v, idx0
        for axis in (1, 2):
            for s in (1, 2, 4):
                for sg in (s, -s):
                    bv = _shift(mv, sg, axis, _NEG)
                    bi = _shift(mi, sg, axis, _BIGI)
                    tb = (bv > mv) | ((bv == mv) & (bi < mi))
                    mv = jnp.where(tb, bv, mv)
                    mi = jnp.where(tb, bi, mi)
        active = v >= 0
        k = active & (mv == v) & (mi == idx0)
        keep_ref[...] = jnp.maximum(keep_ref[...], k.astype(jnp.float32))
        d = k.astype(jnp.float32)
        for axis in (1, 2):
            for s in (1, 2, 4):
                for sg in (s, -s):
                    d = jnp.maximum(d, _shift(d, sg, axis, 0.0))
        vn = jnp.where(d > 0, _NEG, v)
        v_ref[...] = vn
        return vn

    for _ in range(3):
        run_round()

    def round_body(_):
        return jnp.max(run_round()) >= 0

    lax.while_loop(lambda cont: cont, round_body,
                   jnp.max(v_ref[...]) >= 0)

    out_ref[...] = x * keep_ref[...]


def kernel(input_):
    x = input_.reshape(_B, _H, _W)
    out = pl.pallas_call(
        _nms_kernel,
        out_shape=jax.ShapeDtypeStruct((_B, _H, _W), jnp.float32),
        scratch_shapes=[
            pltpu.VMEM((_B, _H, _W), jnp.float32),
            pltpu.VMEM((_B, _H, _W), jnp.float32),
        ],
    )(x)
    return out.reshape(input_.shape)


# speedup vs baseline: 1.0013x; 1.0013x over previous
import jax
import jax.numpy as jnp
from jax import lax
from jax.experimental import pallas as pl
from jax.experimental.pallas import tpu as pltpu

_B, _H, _W = 4, 224, 224
_NEG = -1e30
_BIGI = 1 << 30


def _shift(a, s, axis, fill):
    b, h, w = a.shape
    if axis == 1:
        pad_shape = (b, abs(s), w)
    else:
        pad_shape = (b, h, abs(s))
    pad = jnp.full(pad_shape, fill, a.dtype)
    if axis == 1:
        if s > 0:
            return jnp.concatenate([a[:, s:, :], pad], axis=1)
        return jnp.concatenate([pad, a[:, :s, :]], axis=1)
    else:
        if s > 0:
            return jnp.concatenate([a[:, :, s:], pad], axis=2)
        return jnp.concatenate([pad, a[:, :, :s]], axis=2)


def _nms_kernel(x_ref, out_ref, v_ref, keep_ref):
    x = x_ref[...]

    xr = jnp.concatenate([x[:, :, 1:], x[:, :, -1:]], axis=2)
    xl = jnp.concatenate([x[:, :, :1], x[:, :, :-1]], axis=2)
    xd = jnp.concatenate([x[:, 1:, :], x[:, -1:, :]], axis=1)
    xu = jnp.concatenate([x[:, :1, :], x[:, :-1, :]], axis=1)
    rg_x = xr > x
    ll_x = x <= xl
    rg_y = xd > x
    ll_y = x <= xu
    neg = x <= 0
    valleys = rg_x & ll_x & rg_y & ll_y & neg
    peaks = (~rg_x) & (~ll_x) & (~rg_y) & (~ll_y) & (~neg)
    ext = peaks | valleys

    v_ref[...] = jnp.where(ext, jnp.abs(x), _NEG)
    keep_ref[...] = jnp.zeros_like(x)

    ri = lax.broadcasted_iota(jnp.int32, (_B, _H, _W), 1)
    ci = lax.broadcasted_iota(jnp.int32, (_B, _H, _W), 2)
    idx0 = ri * _W + ci

    def round_body(_):
        v = v_ref[...]
        mv, mi = v, idx0
        for axis in (1, 2):
            for s in (1, 2, 4):
                for sg in (s, -s):
                    bv = _shift(mv, sg, axis, _NEG)
                    bi = _shift(mi, sg, axis, _BIGI)
                    tb = (bv > mv) | ((bv == mv) & (bi < mi))
                    mv = jnp.where(tb, bv, mv)
                    mi = jnp.where(tb, bi, mi)
        active = v >= 0
        k = active & (mv == v) & (mi == idx0)
        kf = k.astype(jnp.float32)
        keep_ref[...] = jnp.maximum(keep_ref[...], kf)
        d = kf
        for axis in (1, 2):
            for s in (1, 2, 4):
                for sg in (s, -s):
                    d = jnp.maximum(d, _shift(d, sg, axis, 0.0))
        vn = jnp.where(d > 0, _NEG, v)
        v_ref[...] = vn
        return jnp.max(vn) >= 0

    lax.while_loop(lambda cont: cont, round_body,
                   jnp.max(v_ref[...]) >= 0)

    out_ref[...] = x * keep_ref[...]


def kernel(input_):
    x = input_.reshape(_B, _H, _W)
    out = pl.pallas_call(
        _nms_kernel,
        out_shape=jax.ShapeDtypeStruct((_B, _H, _W), jnp.float32),
        scratch_shapes=[
            pltpu.VMEM((_B, _H, _W), jnp.float32),
            pltpu.VMEM((_B, _H, _W), jnp.float32),
        ],
    )(x)
    return out.reshape(input_.shape)


# speedup vs baseline: 1.0220x; 1.0207x over previous
import jax
import jax.numpy as jnp
from jax import lax
from jax.experimental import pallas as pl
from jax.experimental.pallas import tpu as pltpu

_B, _H, _W = 4, 224, 224
_NEG = -1e30
_BIGI = 32767


def _shift(a, s, axis, fill):
    b, h, w = a.shape
    if axis == 1:
        pad_shape = (b, abs(s), w)
    else:
        pad_shape = (b, h, abs(s))
    pad = jnp.full(pad_shape, fill, a.dtype)
    if axis == 1:
        if s > 0:
            return jnp.concatenate([a[:, s:, :], pad], axis=1)
        return jnp.concatenate([pad, a[:, :s, :]], axis=1)
    else:
        if s > 0:
            return jnp.concatenate([a[:, :, s:], pad], axis=2)
        return jnp.concatenate([pad, a[:, :, :s]], axis=2)


def _nms_kernel(x_ref, out_ref, v_ref, keep_ref):
    x = x_ref[...]

    xr = jnp.concatenate([x[:, :, 1:], x[:, :, -1:]], axis=2)
    xl = jnp.concatenate([x[:, :, :1], x[:, :, :-1]], axis=2)
    xd = jnp.concatenate([x[:, 1:, :], x[:, -1:, :]], axis=1)
    xu = jnp.concatenate([x[:, :1, :], x[:, :-1, :]], axis=1)
    rg_x = xr > x
    ll_x = x <= xl
    rg_y = xd > x
    ll_y = x <= xu
    neg = x <= 0
    valleys = rg_x & ll_x & rg_y & ll_y & neg
    peaks = (~rg_x) & (~ll_x) & (~rg_y) & (~ll_y) & (~neg)
    ext = peaks | valleys

    v_ref[...] = jnp.where(ext, jnp.abs(x), _NEG)
    keep_ref[...] = jnp.zeros_like(x)

    ri = lax.broadcasted_iota(jnp.int32, (_B, _H, _W), 1)
    ci = lax.broadcasted_iota(jnp.int32, (_B, _H, _W), 2)
    idx0 = (ri * _W + ci - 32768).astype(jnp.int16)

    def round_body(_):
        v = v_ref[...]
        mv, mi = v, idx0
        for axis in (1, 2):
            for s in (1, 2, 4):
                for sg in (s, -s):
                    bv = _shift(mv, sg, axis, _NEG)
                    bi = _shift(mi, sg, axis, _BIGI)
                    tb = (bv > mv) | ((bv == mv) & (bi < mi))
                    mv = jnp.where(tb, bv, mv)
                    mi = jnp.where(tb, bi, mi)
        active = v >= 0
        k = active & (mv == v) & (mi == idx0)
        kf = k.astype(jnp.float32)
        keep_ref[...] = jnp.maximum(keep_ref[...], kf)
        d = kf
        for axis in (1, 2):
            for s in (1, 2, 4):
                for sg in (s, -s):
                    d = jnp.maximum(d, _shift(d, sg, axis, 0.0))
        vn = jnp.where(d > 0, _NEG, v)
        v_ref[...] = vn
        return jnp.max(vn) >= 0

    lax.while_loop(lambda cont: cont, round_body,
                   jnp.max(v_ref[...]) >= 0)

    out_ref[...] = x * keep_ref[...]


def kernel(input_):
    x = input_.reshape(_B, _H, _W)
    out = pl.pallas_call(
        _nms_kernel,
        out_shape=jax.ShapeDtypeStruct((_B, _H, _W), jnp.float32),
        scratch_shapes=[
            pltpu.VMEM((_B, _H, _W), jnp.float32),
            pltpu.VMEM((_B, _H, _W), jnp.float32),
        ],
    )(x)
    return out.reshape(input_.shape)


# speedup vs baseline: 1.0809x; 1.0576x over previous
import jax
import jax.numpy as jnp
from jax import lax
from jax.experimental import pallas as pl
from jax.experimental.pallas import tpu as pltpu

_B, _H, _W = 4, 224, 224
_NEG = -1e30
_BIGI = 32767


def _shift(a, s, axis, fill):
    b, h, w = a.shape
    if axis == 1:
        pad_shape = (b, abs(s), w)
    else:
        pad_shape = (b, h, abs(s))
    pad = jnp.full(pad_shape, fill, a.dtype)
    if axis == 1:
        if s > 0:
            return jnp.concatenate([a[:, s:, :], pad], axis=1)
        return jnp.concatenate([pad, a[:, :s, :]], axis=1)
    else:
        if s > 0:
            return jnp.concatenate([a[:, :, s:], pad], axis=2)
        return jnp.concatenate([pad, a[:, :, :s]], axis=2)


def _nms_kernel(x_ref, out_ref, v_ref, keep_ref):
    x = x_ref[...]

    xr = jnp.concatenate([x[:, :, 1:], x[:, :, -1:]], axis=2)
    xl = jnp.concatenate([x[:, :, :1], x[:, :, :-1]], axis=2)
    xd = jnp.concatenate([x[:, 1:, :], x[:, -1:, :]], axis=1)
    xu = jnp.concatenate([x[:, :1, :], x[:, :-1, :]], axis=1)
    rg_x = xr > x
    ll_x = x <= xl
    rg_y = xd > x
    ll_y = x <= xu
    neg = x <= 0
    valleys = rg_x & ll_x & rg_y & ll_y & neg
    peaks = (~rg_x) & (~ll_x) & (~rg_y) & (~ll_y) & (~neg)
    ext = peaks | valleys

    v_ref[...] = jnp.where(ext, jnp.abs(x), _NEG)
    keep_ref[...] = jnp.zeros_like(x)

    ri = lax.broadcasted_iota(jnp.int32, (_B, _H, _W), 1)
    ci = lax.broadcasted_iota(jnp.int32, (_B, _H, _W), 2)
    idx0 = (ri * _W + ci - 32768).astype(jnp.int16)

    def round_body(_):
        v = v_ref[...]
        mv, mi = v, idx0
        for axis in (1, 2):
            for s in (1, 2, 4):
                for sg in (s, -s):
                    bv = _shift(mv, sg, axis, _NEG)
                    bi = _shift(mi, sg, axis, _BIGI)
                    tb = (bv > mv) | ((bv == mv) & (bi < mi))
                    mv = jnp.where(tb, bv, mv)
                    mi = jnp.where(tb, bi, mi)
        active = v >= 0
        k = active & (mv == v) & (mi == idx0)
        keep_ref[...] = jnp.maximum(keep_ref[...], k.astype(jnp.float32))
        d = k.astype(jnp.bfloat16)
        for axis in (1, 2):
            for s in (1, 2, 4):
                for sg in (s, -s):
                    d = jnp.maximum(d, _shift(d, sg, axis, 0.0))
        vn = jnp.where(d > 0, _NEG, v)
        v_ref[...] = vn
        return jnp.max(vn) >= 0

    lax.while_loop(lambda cont: cont, round_body,
                   jnp.max(v_ref[...]) >= 0)

    out_ref[...] = x * keep_ref[...]


def kernel(input_):
    x = input_.reshape(_B, _H, _W)
    out = pl.pallas_call(
        _nms_kernel,
        out_shape=jax.ShapeDtypeStruct((_B, _H, _W), jnp.float32),
        scratch_shapes=[
            pltpu.VMEM((_B, _H, _W), jnp.float32),
            pltpu.VMEM((_B, _H, _W), jnp.float32),
        ],
    )(x)
    return out.reshape(input_.shape)
